# baseline (speedup 1.0000x reference)
import functools

import jax
import jax.numpy as jnp
from jax import lax
from jax.experimental import pallas as pl
from jax.experimental.pallas import tpu as pltpu
from jax.experimental.pallas import tpu_sc as plsc

N = 10000
D = 128
NC = 40
LAMB = 1.5

NSC = 2
NSUB = 16
NW = NSC * NSUB
CHUNK = 64
CH = 160
GROUPS = 4
G_CH = CH // GROUPS
E_PAD = NW * CH * CHUNK
NBUF = 4
N_ACC = 10240
ROWS_PER_TILE = N_ACC // NSUB

BLK = 2000
GRID = N // BLK


def _irelu(x):
    return jnp.maximum(0.0, jnp.minimum(LAMB * x, 1.0 + x / LAMB))


def _mm(a, w):
    return lax.dot_general(a, w, (((1,), (1,)), ((), ())),
                           preferred_element_type=jnp.float32)


def _rowspec(d=D):
    return pl.BlockSpec((BLK, d), lambda i: (i, 0))


def _wspec(shape):
    return pl.BlockSpec(shape, lambda i: (0, 0))


def _pspec(core):
    return pl.BlockSpec((1, BLK, D), lambda i, c=core: (c, i, 0))


def _enc_mlp_body(x_ref, w0, b0, w1, b1, w2, b2, o_ref):
    h = _irelu(_mm(x_ref[...], w0[...]) + b0[...])
    h = _irelu(_mm(h, w1[...]) + b1[...])
    h = _irelu(_mm(h, w2[...]) + b2[...])
    o_ref[...] = h


def _enc_mlp(x, w0, b0, w1, b1, w2, b2):
    return pl.pallas_call(
        _enc_mlp_body,
        grid=(GRID,),
        in_specs=[_rowspec(), _wspec((D, D)), _wspec((1, D)),
                  _wspec((D, D)), _wspec((1, D)),
                  _wspec((D, D)), _wspec((1, D))],
        out_specs=_rowspec(),
        out_shape=jax.ShapeDtypeStruct((N, D), jnp.float32),
    )(x, w0, b0.reshape(1, D), w1, b1.reshape(1, D), w2, b2.reshape(1, D))


def _agg_mlp_body(p0, p1, h_ref, wa, wh, ba, w0, b0, w1, b1, o_ref):
    t = _irelu(_mm(p0[0] + p1[0], wa[...]) + _mm(h_ref[...], wh[...]) + ba[...])
    t = _irelu(_mm(t, w0[...]) + b0[...])
    t = _irelu(_mm(t, w1[...]) + b1[...])
    o_ref[...] = t


def _agg_mlp(p, h, wa, wh, ba, w0, b0, w1, b1):
    return pl.pallas_call(
        _agg_mlp_body,
        grid=(GRID,),
        in_specs=[_pspec(0), _pspec(1),
                  _rowspec(),
                  _wspec((D, D)), _wspec((D, D)), _wspec((1, D)),
                  _wspec((D, D)), _wspec((1, D)),
                  _wspec((D, D)), _wspec((1, D))],
        out_specs=_rowspec(),
        out_shape=jax.ShapeDtypeStruct((N, D), jnp.float32),
    )(p, p, h, wa, wh, ba.reshape(1, D), w0, b0.reshape(1, D), w1, b1.reshape(1, D))


def _agg_dec_body(p0, p1, h_ref, wa, wh, ba, wd, bd, o_ref):
    t = _irelu(_mm(p0[0] + p1[0], wa[...]) + _mm(h_ref[...], wh[...]) + ba[...])
    o_ref[...] = _irelu(_mm(t, wd[...]) + bd[...])


def _agg_dec(p, h, wa, wh, ba, wd, bd):
    return pl.pallas_call(
        _agg_dec_body,
        grid=(GRID,),
        in_specs=[_pspec(0), _pspec(1),
                  _rowspec(),
                  _wspec((D, D)), _wspec((D, D)), _wspec((1, D)),
                  _wspec((NC, D)), _wspec((1, NC))],
        out_specs=_rowspec(NC),
        out_shape=jax.ShapeDtypeStruct((N, NC), jnp.float32),
    )(p, p, h, wa, wh, ba.reshape(1, D), wd, bd.reshape(1, NC))


def _segment_sum_sc(h, src2d, dst2d, zeros_acc):
    mesh = plsc.VectorSubcoreMesh(core_axis_name="c", subcore_axis_name="s")

    @functools.partial(
        pl.kernel,
        mesh=mesh,
        out_type=jax.ShapeDtypeStruct((NSC * N_ACC, D), jnp.float32),
        scratch_types=[
            pltpu.VMEM((G_CH, CHUNK), jnp.int32),
            pltpu.VMEM((G_CH, CHUNK), jnp.int32),
            pltpu.VMEM((NBUF, CHUNK, D), jnp.float32),
            pltpu.VMEM_SHARED((N_ACC, D), jnp.float32),
        ] + [pltpu.SemaphoreType.DMA] * (2 * NBUF),
    )
    def k(h_hbm, src_hbm, dst_hbm, z_hbm, out_hbm, src_v, dst_v, rows_v,
          acc_sh, *sems):
        gsems, ssems = sems[:NBUF], sems[NBUF:]
        c = lax.axis_index("c")
        s = lax.axis_index("s")
        wid = s * NSC + c
        base = wid * CH
        r0 = s * ROWS_PER_TILE
        pltpu.sync_copy(z_hbm.at[pl.ds(r0, ROWS_PER_TILE)],
                        acc_sh.at[pl.ds(r0, ROWS_PER_TILE)])
        plsc.subcore_barrier()

        def body(i, carry):
            j = i * NBUF

            @pl.when(i > 0)
            def _():
                for b in range(NBUF):
                    pltpu.make_async_copy(
                        rows_v.at[b], acc_sh.at[dst_v.at[j - NBUF + b]],
                        ssems[b]).wait()

            cps = [pltpu.async_copy(h_hbm.at[src_v.at[j + b]], rows_v.at[b],
                                    gsems[b])
                   for b in range(NBUF)]
            for b in range(NBUF):
                cps[b].wait()
                pltpu.async_copy(rows_v.at[b], acc_sh.at[dst_v.at[j + b]],
                                 ssems[b], add=True)
            return carry

        for g in range(GROUPS):
            pltpu.sync_copy(src_hbm.at[pl.ds(base + g * G_CH, G_CH)], src_v)
            pltpu.sync_copy(dst_hbm.at[pl.ds(base + g * G_CH, G_CH)], dst_v)
            lax.fori_loop(0, G_CH // NBUF, body, 0)
            for b in range(NBUF):
                pltpu.make_async_copy(
                    rows_v.at[b], acc_sh.at[dst_v.at[G_CH - NBUF + b]],
                    ssems[b]).wait()
        plsc.subcore_barrier()
        off = c * N_ACC + r0
        pltpu.sync_copy(acc_sh.at[pl.ds(r0, ROWS_PER_TILE)],
                        out_hbm.at[pl.ds(off, ROWS_PER_TILE)])

    return k(h, src2d, dst2d, zeros_acc)


def kernel(x, edge_index, enc_W, enc_b, b0_l0_W, b0_l0_b, b0_l1_W, b0_l1_b,
           b0_agg_W, b0_agg_b, b1_l0_W, b1_l0_b, b1_l1_W, b1_l1_b,
           b1_agg_W, b1_agg_b, dec_W, dec_b):
    src = edge_index[0]
    dst = edge_index[1]
    pad = E_PAD - src.shape[0]
    ar = jnp.arange(pad, dtype=jnp.int32)
    src2d = jnp.concatenate([src, (ar * 97) % N]).reshape(NW * CH, CHUNK)
    dst2d = jnp.concatenate([dst, N + ar % (N_ACC - N)]).reshape(NW * CH, CHUNK)
    zeros_acc = jnp.zeros((N_ACC, D), jnp.float32)

    wa0, wh0 = b0_agg_W[:, :D], b0_agg_W[:, D:]
    wa1, wh1 = b1_agg_W[:, :D], b1_agg_W[:, D:]

    h = _enc_mlp(x, enc_W, enc_b, b0_l0_W, b0_l0_b, b0_l1_W, b0_l1_b)
    p = _segment_sum_sc(h, src2d, dst2d, zeros_acc).reshape(NSC, N_ACC, D)
    h = _agg_mlp(p, h, wa0, wh0, b0_agg_b, b1_l0_W, b1_l0_b, b1_l1_W, b1_l1_b)
    p = _segment_sum_sc(h, src2d, dst2d, zeros_acc).reshape(NSC, N_ACC, D)
    return _agg_dec(p, h, wa1, wh1, b1_agg_b, dec_W, dec_b)


# speedup vs baseline: 1.0487x; 1.0487x over previous
import functools

import jax
import jax.numpy as jnp
from jax import lax
from jax.experimental import pallas as pl
from jax.experimental.pallas import tpu as pltpu
from jax.experimental.pallas import tpu_sc as plsc

N = 10000
D = 128
NC = 40
LAMB = 1.5

NSC = 2
NSUB = 16
NW = NSC * NSUB
CHUNK = 128
CH = 80
GROUPS = 2
G_CH = CH // GROUPS
E_PAD = NW * CH * CHUNK
NBUF = 2
N_ACC = 10240
ROWS_PER_TILE = N_ACC // NSUB

BLK = 2000
GRID = N // BLK


def _irelu(x):
    return jnp.maximum(0.0, jnp.minimum(LAMB * x, 1.0 + x / LAMB))


def _mm(a, w):
    return lax.dot_general(a, w, (((1,), (1,)), ((), ())),
                           preferred_element_type=jnp.float32)


def _rowspec(d=D):
    return pl.BlockSpec((BLK, d), lambda i: (i, 0))


def _wspec(shape):
    return pl.BlockSpec(shape, lambda i: (0, 0))


def _pspec(core):
    return pl.BlockSpec((1, BLK, D), lambda i, c=core: (c, i, 0))


def _enc_mlp_body(x_ref, w0, b0, w1, b1, w2, b2, o_ref):
    h = _irelu(_mm(x_ref[...], w0[...]) + b0[...])
    h = _irelu(_mm(h, w1[...]) + b1[...])
    h = _irelu(_mm(h, w2[...]) + b2[...])
    o_ref[...] = h


def _enc_mlp(x, w0, b0, w1, b1, w2, b2):
    return pl.pallas_call(
        _enc_mlp_body,
        grid=(GRID,),
        in_specs=[_rowspec(), _wspec((D, D)), _wspec((1, D)),
                  _wspec((D, D)), _wspec((1, D)),
                  _wspec((D, D)), _wspec((1, D))],
        out_specs=_rowspec(),
        out_shape=jax.ShapeDtypeStruct((N, D), jnp.float32),
    )(x, w0, b0.reshape(1, D), w1, b1.reshape(1, D), w2, b2.reshape(1, D))


def _hw_body(h_ref, wh, ba, o_ref):
    o_ref[...] = _mm(h_ref[...], wh[...]) + ba[...]


def _hw(h, wh, ba):
    return pl.pallas_call(
        _hw_body,
        grid=(GRID,),
        in_specs=[_rowspec(), _wspec((D, D)), _wspec((1, D))],
        out_specs=_rowspec(),
        out_shape=jax.ShapeDtypeStruct((N, D), jnp.float32),
    )(h, wh, ba.reshape(1, D))


def _agg_mlp_body(p0, p1, hw_ref, wa, w0, b0, w1, b1, o_ref):
    t = _irelu(_mm(p0[0] + p1[0], wa[...]) + hw_ref[...])
    t = _irelu(_mm(t, w0[...]) + b0[...])
    t = _irelu(_mm(t, w1[...]) + b1[...])
    o_ref[...] = t


def _agg_mlp(p, hw, wa, w0, b0, w1, b1):
    return pl.pallas_call(
        _agg_mlp_body,
        grid=(GRID,),
        in_specs=[_pspec(0), _pspec(1),
                  _rowspec(),
                  _wspec((D, D)),
                  _wspec((D, D)), _wspec((1, D)),
                  _wspec((D, D)), _wspec((1, D))],
        out_specs=_rowspec(),
        out_shape=jax.ShapeDtypeStruct((N, D), jnp.float32),
    )(p, p, hw, wa, w0, b0.reshape(1, D), w1, b1.reshape(1, D))


def _agg_dec_body(p0, p1, hw_ref, wa, wd, bd, o_ref):
    t = _irelu(_mm(p0[0] + p1[0], wa[...]) + hw_ref[...])
    o_ref[...] = _irelu(_mm(t, wd[...]) + bd[...])


def _agg_dec(p, hw, wa, wd, bd):
    return pl.pallas_call(
        _agg_dec_body,
        grid=(GRID,),
        in_specs=[_pspec(0), _pspec(1),
                  _rowspec(),
                  _wspec((D, D)),
                  _wspec((NC, D)), _wspec((1, NC))],
        out_specs=_rowspec(NC),
        out_shape=jax.ShapeDtypeStruct((N, NC), jnp.float32),
    )(p, p, hw, wa, wd, bd.reshape(1, NC))


def _segment_sum_sc(h, src2d, dst2d):
    mesh = plsc.VectorSubcoreMesh(core_axis_name="c", subcore_axis_name="s")

    @functools.partial(
        pl.kernel,
        mesh=mesh,
        out_type=jax.ShapeDtypeStruct((NSC * N_ACC, D), jnp.float32),
        scratch_types=[
            pltpu.VMEM((G_CH, CHUNK), jnp.int32),
            pltpu.VMEM((G_CH, CHUNK), jnp.int32),
            pltpu.VMEM((NBUF, CHUNK, D), jnp.float32),
            pltpu.VMEM_SHARED((N_ACC, D), jnp.float32),
        ] + [pltpu.SemaphoreType.DMA] * (2 * NBUF),
    )
    def k(h_hbm, src_hbm, dst_hbm, out_hbm, src_v, dst_v, rows_v,
          acc_sh, *sems):
        gsems, ssems = sems[:NBUF], sems[NBUF:]
        c = lax.axis_index("c")
        s = lax.axis_index("s")
        wid = s * NSC + c
        base = wid * CH
        zero = jnp.zeros((16,), jnp.float32)

        def zbody(r, carry):
            for jj in range(D // 16):
                rows_v[0, r, pl.ds(jj * 16, 16)] = zero
            return carry

        lax.fori_loop(0, CHUNK, zbody, 0)
        r0 = s * ROWS_PER_TILE
        for t in range(ROWS_PER_TILE // CHUNK):
            pltpu.sync_copy(rows_v.at[0],
                            acc_sh.at[pl.ds(r0 + t * CHUNK, CHUNK)])
        plsc.subcore_barrier()

        def body(i, carry):
            j = i * NBUF

            @pl.when(i > 0)
            def _():
                for b in range(NBUF):
                    pltpu.make_async_copy(
                        rows_v.at[b], acc_sh.at[dst_v.at[j - NBUF + b]],
                        ssems[b]).wait()

            cps = [pltpu.async_copy(h_hbm.at[src_v.at[j + b]], rows_v.at[b],
                                    gsems[b])
                   for b in range(NBUF)]
            for b in range(NBUF):
                cps[b].wait()
                pltpu.async_copy(rows_v.at[b], acc_sh.at[dst_v.at[j + b]],
                                 ssems[b], add=True)
            return carry

        for g in range(GROUPS):
            pltpu.sync_copy(src_hbm.at[pl.ds(base + g * G_CH, G_CH)], src_v)
            pltpu.sync_copy(dst_hbm.at[pl.ds(base + g * G_CH, G_CH)], dst_v)
            lax.fori_loop(0, G_CH // NBUF, body, 0)
            for b in range(NBUF):
                pltpu.make_async_copy(
                    rows_v.at[b], acc_sh.at[dst_v.at[G_CH - NBUF + b]],
                    ssems[b]).wait()
        plsc.subcore_barrier()
        off = c * N_ACC + r0
        pltpu.sync_copy(acc_sh.at[pl.ds(r0, ROWS_PER_TILE)],
                        out_hbm.at[pl.ds(off, ROWS_PER_TILE)])

    return k(h, src2d, dst2d)


def kernel(x, edge_index, enc_W, enc_b, b0_l0_W, b0_l0_b, b0_l1_W, b0_l1_b,
           b0_agg_W, b0_agg_b, b1_l0_W, b1_l0_b, b1_l1_W, b1_l1_b,
           b1_agg_W, b1_agg_b, dec_W, dec_b):
    src = edge_index[0]
    dst = edge_index[1]
    pad = E_PAD - src.shape[0]
    ar = jnp.arange(pad, dtype=jnp.int32)
    src2d = jnp.concatenate([src, (ar * 97) % N]).reshape(NW * CH, CHUNK)
    dst2d = jnp.concatenate([dst, N + ar % (N_ACC - N)]).reshape(NW * CH, CHUNK)

    wa0, wh0 = b0_agg_W[:, :D], b0_agg_W[:, D:]
    wa1, wh1 = b1_agg_W[:, :D], b1_agg_W[:, D:]

    h = _enc_mlp(x, enc_W, enc_b, b0_l0_W, b0_l0_b, b0_l1_W, b0_l1_b)
    p = _segment_sum_sc(h, src2d, dst2d).reshape(NSC, N_ACC, D)
    hw = _hw(h, wh0, b0_agg_b)
    h = _agg_mlp(p, hw, wa0, b1_l0_W, b1_l0_b, b1_l1_W, b1_l1_b)
    p = _segment_sum_sc(h, src2d, dst2d).reshape(NSC, N_ACC, D)
    hw = _hw(h, wh1, b1_agg_b)
    return _agg_dec(p, hw, wa1, dec_W, dec_b)
